# Initial kernel scaffold; baseline (speedup 1.0000x reference)
import jax
import jax.numpy as jnp
from jax.experimental import pallas as pl


def kernel(features, nodes, to_neighs, num_sample):
    raise NotImplementedError("write your pallas kernel here")


# speedup vs baseline: 4.1674x; 4.1674x over previous
import functools

import jax
import jax.numpy as jnp
from jax import lax
from jax.experimental import pallas as pl
from jax.experimental.pallas import tpu as pltpu
from jax.experimental.pallas import tpu_sc as plsc

NC = 2
NS = 16
NW = NC * NS
LANES = 16

C = 40
IDX_W = 80


def _agg_body(S, D, NCH, features_hbm, tn_hbm, scale_hbm, out_hbm,
              idx_v, rows_v, out_v, scale_v, sem):
    k_slices = (C * S) // IDX_W
    d_regs = D // LANES

    wid = lax.axis_index("s") * NC + lax.axis_index("c")
    n_chunks = (NCH - wid + NW - 1) // NW

    pltpu.sync_copy(scale_hbm, scale_v)
    scale = scale_v[...]

    def chunk_body(i, _):
        c = wid + i * NW
        pltpu.sync_copy(tn_hbm.at[c], idx_v)
        copies = [
            pltpu.make_async_copy(
                features_hbm.at[idx_v.at[j]],
                rows_v.at[pl.ds(j * IDX_W, IDX_W)], sem)
            for j in range(k_slices)
        ]
        for cp in copies:
            cp.start()
        for cp in copies:
            cp.wait()

        def node_body(n, _):
            r = n * S
            for k in range(d_regs):
                sl = pl.ds(k * LANES, LANES)
                acc = rows_v[r, sl]
                for s in range(1, S):
                    acc = acc + rows_v[r + s, sl]
                out_v[n, sl] = acc * scale
            return 0

        lax.fori_loop(0, C, node_body, 0, unroll=False)

        pltpu.sync_copy(out_v, out_hbm.at[pl.ds(c * C, C)])
        return 0

    lax.fori_loop(0, n_chunks, chunk_body, 0, unroll=False)


def kernel(features, nodes, to_neighs, num_sample):
    del nodes
    B, S = to_neighs.shape
    N, D = features.shape
    assert B % C == 0 and (C * S) % IDX_W == 0 and D % LANES == 0
    NCH = B // C

    tn = to_neighs.reshape(NCH, (C * S) // IDX_W, IDX_W)
    scale = jnp.full((LANES,), 1.0, jnp.float32) / num_sample

    mesh = plsc.VectorSubcoreMesh(
        core_axis_name="c", subcore_axis_name="s",
        num_cores=NC, num_subcores=NS)
    grid_kernel = functools.partial(
        pl.kernel,
        out_type=jax.ShapeDtypeStruct((B, D), jnp.float32),
        mesh=mesh,
        scratch_types=[
            pltpu.VMEM(((C * S) // IDX_W, IDX_W), jnp.int32),
            pltpu.VMEM((C * S, D), jnp.float32),
            pltpu.VMEM((C, D), jnp.float32),
            pltpu.VMEM((LANES,), jnp.float32),
            pltpu.SemaphoreType.DMA,
        ],
    )(functools.partial(_agg_body, S, D, NCH))
    return grid_kernel(features, tn, scale)


# speedup vs baseline: 5.7085x; 1.3698x over previous
import functools

import jax
import jax.numpy as jnp
from jax import lax
from jax.experimental import pallas as pl
from jax.experimental.pallas import tpu as pltpu
from jax.experimental.pallas import tpu_sc as plsc

NC = 2
NS = 16
NW = NC * NS
LANES = 16

C = 40
IDX_W = 80


def _agg_body(S, D, NCH, features_hbm, tn_hbm, scale_hbm, out_hbm,
              idx0, idx1, rows0, rows1, out_v, scale_v, sem0, sem1):
    k_slices = (C * S) // IDX_W
    d_regs = D // LANES

    wid = lax.axis_index("s") * NC + lax.axis_index("c")

    pltpu.sync_copy(scale_hbm, scale_v)
    scale = scale_v[...]

    idx = (idx0, idx1)
    rows = (rows0, rows1)
    sems = (sem0, sem1)

    def gather_copies(p):
        return [
            pltpu.make_async_copy(
                features_hbm.at[idx[p].at[j]],
                rows[p].at[pl.ds(j * IDX_W, IDX_W)], sems[p])
            for j in range(k_slices)
        ]

    def start(c, p):
        pltpu.sync_copy(tn_hbm.at[c], idx[p])
        for cp in gather_copies(p):
            cp.start()

    def finish(c, p):
        for cp in gather_copies(p):
            cp.wait()

        def node_body(n, _):
            r = n * S
            for k in range(d_regs):
                sl = pl.ds(k * LANES, LANES)
                acc = rows[p][r, sl]
                for s in range(1, S):
                    acc = acc + rows[p][r + s, sl]
                out_v[n, sl] = acc * scale
            return 0

        lax.fori_loop(0, C, node_body, 0, unroll=False)
        pltpu.sync_copy(out_v, out_hbm.at[pl.ds(c * C, C)])

    @pl.when(wid < NCH)
    def _():
        start(wid, 0)

    n_pairs = (-(-NCH // NW) + 1) // 2
    def pair_body(i2, _):
        for p in (0, 1):
            t = 2 * i2 + p
            c = wid + t * NW

            @pl.when(c + NW < NCH)
            def _():
                start(c + NW, 1 - p)

            @pl.when(c < NCH)
            def _():
                finish(c, p)
        return 0

    lax.fori_loop(0, n_pairs, pair_body, 0, unroll=False)


def kernel(features, nodes, to_neighs, num_sample):
    del nodes
    B, S = to_neighs.shape
    N, D = features.shape
    assert B % C == 0 and (C * S) % IDX_W == 0 and D % LANES == 0
    NCH = B // C

    tn = to_neighs.reshape(NCH, (C * S) // IDX_W, IDX_W)
    scale = jnp.full((LANES,), 1.0, jnp.float32) / num_sample

    mesh = plsc.VectorSubcoreMesh(
        core_axis_name="c", subcore_axis_name="s",
        num_cores=NC, num_subcores=NS)
    k_sl = (C * S) // IDX_W
    grid_kernel = functools.partial(
        pl.kernel,
        out_type=jax.ShapeDtypeStruct((B, D), jnp.float32),
        mesh=mesh,
        scratch_types=[
            pltpu.VMEM((k_sl, IDX_W), jnp.int32),
            pltpu.VMEM((k_sl, IDX_W), jnp.int32),
            pltpu.VMEM((C * S, D), jnp.float32),
            pltpu.VMEM((C * S, D), jnp.float32),
            pltpu.VMEM((C, D), jnp.float32),
            pltpu.VMEM((LANES,), jnp.float32),
            pltpu.SemaphoreType.DMA,
            pltpu.SemaphoreType.DMA,
        ],
    )(functools.partial(_agg_body, S, D, NCH))
    return grid_kernel(features, tn, scale)


# speedup vs baseline: 6.5214x; 1.1424x over previous
import functools

import jax
import jax.numpy as jnp
from jax import lax
from jax.experimental import pallas as pl
from jax.experimental.pallas import tpu as pltpu
from jax.experimental.pallas import tpu_sc as plsc

NC = 2
NS = 16
NW = NC * NS
LANES = 16

C = 40
IDX_W = 80


def _agg_body(S, D, NCH, features_hbm, tn_hbm, scale_hbm, out_hbm,
              idx0, idx1, rows0, rows1, out0, out1, scale_v,
              gsem0, gsem1, isem0, isem1, osem0, osem1):
    k_slices = (C * S) // IDX_W
    d_regs = D // LANES

    wid = lax.axis_index("s") * NC + lax.axis_index("c")

    pltpu.sync_copy(scale_hbm, scale_v)
    scale = scale_v[...]

    idx = (idx0, idx1)
    rows = (rows0, rows1)
    outs = (out0, out1)
    gsem = (gsem0, gsem1)
    isem = (isem0, isem1)
    osem = (osem0, osem1)

    def idx_copy(c, p):
        return pltpu.make_async_copy(tn_hbm.at[c], idx[p], isem[p])

    def gather_copies(p):
        return [
            pltpu.make_async_copy(
                features_hbm.at[idx[p].at[j]],
                rows[p].at[pl.ds(j * IDX_W, IDX_W)], gsem[p])
            for j in range(k_slices)
        ]

    def out_copy(c, p):
        return pltpu.make_async_copy(outs[p], out_hbm.at[pl.ds(c * C, C)],
                                     osem[p])

    def compute(p):
        def node_body(n, _):
            r = n * S
            for k in range(d_regs):
                sl = pl.ds(k * LANES, LANES)
                acc = rows[p][r, sl]
                for s in range(1, S):
                    acc = acc + rows[p][r + s, sl]
                outs[p][n, sl] = acc * scale
            return 0

        lax.fori_loop(0, C, node_body, 0, unroll=2)

    @pl.when(wid < NCH)
    def _():
        idx_copy(wid, 0).start()
        idx_copy(wid, 0).wait()
        for cp in gather_copies(0):
            cp.start()

    @pl.when(wid + NW < NCH)
    def _():
        idx_copy(wid + NW, 1).start()

    n_pairs = (-(-NCH // NW) + 1) // 2

    def pair_body(i2, _):
        for p in (0, 1):
            t = 2 * i2 + p
            c = wid + t * NW
            q = 1 - p

            @pl.when(c + NW < NCH)
            def _():
                idx_copy(c + NW, q).wait()
                for cp in gather_copies(q):
                    cp.start()

            @pl.when(c < NCH)
            def _():
                for cp in gather_copies(p):
                    cp.wait()

                @pl.when(c + 2 * NW < NCH)
                def _():
                    idx_copy(c + 2 * NW, p).start()

                @pl.when(i2 >= 1)
                def _():
                    out_copy(c - 2 * NW, p).wait()

                compute(p)
                out_copy(c, p).start()
        return 0

    lax.fori_loop(0, n_pairs, pair_body, 0, unroll=False)

    for p in (0, 1):
        @pl.when(wid + p * NW < NCH)
        def _():
            out_copy(0, p).wait()


def kernel(features, nodes, to_neighs, num_sample):
    del nodes
    B, S = to_neighs.shape
    N, D = features.shape
    assert B % C == 0 and (C * S) % IDX_W == 0 and D % LANES == 0
    NCH = B // C

    tn = to_neighs.reshape(NCH, (C * S) // IDX_W, IDX_W)
    scale = jnp.full((LANES,), 1.0, jnp.float32) / num_sample

    mesh = plsc.VectorSubcoreMesh(
        core_axis_name="c", subcore_axis_name="s",
        num_cores=NC, num_subcores=NS)
    k_sl = (C * S) // IDX_W
    grid_kernel = functools.partial(
        pl.kernel,
        out_type=jax.ShapeDtypeStruct((B, D), jnp.float32),
        mesh=mesh,
        scratch_types=[
            pltpu.VMEM((k_sl, IDX_W), jnp.int32),
            pltpu.VMEM((k_sl, IDX_W), jnp.int32),
            pltpu.VMEM((C * S, D), jnp.float32),
            pltpu.VMEM((C * S, D), jnp.float32),
            pltpu.VMEM((C, D), jnp.float32),
            pltpu.VMEM((C, D), jnp.float32),
            pltpu.VMEM((LANES,), jnp.float32),
            pltpu.SemaphoreType.DMA,
            pltpu.SemaphoreType.DMA,
            pltpu.SemaphoreType.DMA,
            pltpu.SemaphoreType.DMA,
            pltpu.SemaphoreType.DMA,
            pltpu.SemaphoreType.DMA,
        ],
    )(functools.partial(_agg_body, S, D, NCH))
    return grid_kernel(features, tn, scale)


# speedup vs baseline: 10.8282x; 1.6604x over previous
import functools

import jax
import jax.numpy as jnp
from jax import lax
from jax.experimental import pallas as pl
from jax.experimental.pallas import tpu as pltpu
from jax.experimental.pallas import tpu_sc as plsc

NC = 2
NS = 16
NW = NC * NS
LANES = 16

C = 40
IDX_W = 80


def _agg_body(S, D, NCH, features_hbm, tn_hbm, scale_hbm, out_hbm,
              idx0, idx1, rows0, rows1, out0, out1, scale_v,
              gsem0, gsem1, isem0, isem1, osem0, osem1):
    k_slices = (C * S) // IDX_W
    d_regs = D // LANES

    wid = lax.axis_index("s") * NC + lax.axis_index("c")

    pltpu.sync_copy(scale_hbm, scale_v)
    scale = scale_v[...]

    idx = (idx0, idx1)
    rows = (rows0, rows1)
    outs = (out0, out1)
    gsem = (gsem0, gsem1)
    isem = (isem0, isem1)
    osem = (osem0, osem1)

    def idx_copy(c, p):
        return pltpu.make_async_copy(tn_hbm.at[c], idx[p], isem[p])

    def gather_copies(p):
        return [
            pltpu.make_async_copy(
                features_hbm.at[idx[p].at[j]],
                rows[p].at[pl.ds(j * IDX_W, IDX_W)], gsem[p])
            for j in range(k_slices)
        ]

    def out_copy(c, p):
        return pltpu.make_async_copy(outs[p], out_hbm.at[pl.ds(c * C, C)],
                                     osem[p])

    def compute(p):
        @plsc.parallel_loop(0, C, unroll=4)
        def node_body(n):
            r = n * S
            for k in range(d_regs):
                sl = pl.ds(k * LANES, LANES)
                acc = rows[p][r, sl]
                for s in range(1, S):
                    acc = acc + rows[p][r + s, sl]
                outs[p][n, sl] = acc * scale

    @pl.when(wid < NCH)
    def _():
        idx_copy(wid, 0).start()
        idx_copy(wid, 0).wait()
        for cp in gather_copies(0):
            cp.start()

    @pl.when(wid + NW < NCH)
    def _():
        idx_copy(wid + NW, 1).start()

    n_pairs = (-(-NCH // NW) + 1) // 2

    def pair_body(i2, _):
        for p in (0, 1):
            t = 2 * i2 + p
            c = wid + t * NW
            q = 1 - p

            @pl.when(c + NW < NCH)
            def _():
                idx_copy(c + NW, q).wait()
                for cp in gather_copies(q):
                    cp.start()

            @pl.when(c < NCH)
            def _():
                for cp in gather_copies(p):
                    cp.wait()

                @pl.when(c + 2 * NW < NCH)
                def _():
                    idx_copy(c + 2 * NW, p).start()

                @pl.when(i2 >= 1)
                def _():
                    out_copy(c - 2 * NW, p).wait()

                compute(p)
                out_copy(c, p).start()
        return 0

    lax.fori_loop(0, n_pairs, pair_body, 0, unroll=False)

    for p in (0, 1):
        @pl.when(wid + p * NW < NCH)
        def _():
            out_copy(0, p).wait()


def kernel(features, nodes, to_neighs, num_sample):
    del nodes
    B, S = to_neighs.shape
    N, D = features.shape
    assert B % C == 0 and (C * S) % IDX_W == 0 and D % LANES == 0
    NCH = B // C

    tn = to_neighs.reshape(NCH, (C * S) // IDX_W, IDX_W)
    scale = jnp.full((LANES,), 1.0, jnp.float32) / num_sample

    mesh = plsc.VectorSubcoreMesh(
        core_axis_name="c", subcore_axis_name="s",
        num_cores=NC, num_subcores=NS)
    k_sl = (C * S) // IDX_W
    grid_kernel = functools.partial(
        pl.kernel,
        out_type=jax.ShapeDtypeStruct((B, D), jnp.float32),
        mesh=mesh,
        scratch_types=[
            pltpu.VMEM((k_sl, IDX_W), jnp.int32),
            pltpu.VMEM((k_sl, IDX_W), jnp.int32),
            pltpu.VMEM((C * S, D), jnp.float32),
            pltpu.VMEM((C * S, D), jnp.float32),
            pltpu.VMEM((C, D), jnp.float32),
            pltpu.VMEM((C, D), jnp.float32),
            pltpu.VMEM((LANES,), jnp.float32),
            pltpu.SemaphoreType.DMA,
            pltpu.SemaphoreType.DMA,
            pltpu.SemaphoreType.DMA,
            pltpu.SemaphoreType.DMA,
            pltpu.SemaphoreType.DMA,
            pltpu.SemaphoreType.DMA,
        ],
    )(functools.partial(_agg_body, S, D, NCH))
    return grid_kernel(features, tn, scale)
